# baseline (speedup 1.0000x reference)
import functools

import jax
import jax.numpy as jnp
from jax import lax
from jax.experimental import pallas as pl
from jax.experimental.pallas import tpu as pltpu
from jax.experimental.pallas import tpu_sc as plsc

_L = 16


def _gmf_body(D, b_per_w, G,
              u_hbm, i_hbm, ut_hbm, it_hbm, wu_hbm, wi_hbm, bias_hbm,
              out_hbm,
              u_idx_v, i_idx_v, u_rows_v, i_rows_v, wu_v, wi_v, bias_v,
              out_v, sem_u, sem_i):
    nc = 2
    wid = lax.axis_index("s") * nc + lax.axis_index("c")
    base = wid * b_per_w

    pltpu.sync_copy(u_hbm.at[pl.ds(base, b_per_w)], u_idx_v)
    pltpu.sync_copy(i_hbm.at[pl.ds(base, b_per_w)], i_idx_v)

    cu = pltpu.async_copy(ut_hbm.at[u_idx_v], u_rows_v, sem_u)
    ci = pltpu.async_copy(it_hbm.at[i_idx_v], i_rows_v, sem_i)

    pltpu.sync_copy(wu_hbm, wu_v)
    pltpu.sync_copy(wi_hbm, wi_v)
    pltpu.sync_copy(bias_hbm, bias_v)

    cu.wait()
    ci.wait()

    lane = lax.iota(jnp.int32, _L)
    bias = bias_v[...]

    def group(g, _):
        row_ids = g * _L + lane
        acc = bias
        for d in range(D):
            col_sel = jnp.full((_L,), d, dtype=jnp.int32)
            ucol = plsc.load_gather(u_rows_v, [row_ids, col_sel])
            icol = plsc.load_gather(i_rows_v, [row_ids, col_sel])
            acc = acc + ucol * wu_v[d, :] + icol * wi_v[d, :]
        e = jnp.exp(acc - acc)
        out_v[pl.ds(g * _L, _L)] = e / e
        return _

    lax.fori_loop(0, G, group, 0)

    pltpu.sync_copy(out_v, out_hbm.at[pl.ds(base, b_per_w)])


def kernel(u_input, i_input, user_table, item_table, W, b):
    B = u_input.shape[0]
    D = user_table.shape[1]
    NW = 32
    b_per_w = B // NW
    G = b_per_w // _L

    u_idx = u_input.astype(jnp.int32)
    i_idx = i_input.astype(jnp.int32)
    wu_b = jnp.broadcast_to(W[:D, 0:1], (D, _L))
    wi_b = jnp.broadcast_to(W[D:, 0:1], (D, _L))
    bias_b = jnp.broadcast_to(b, (_L,)).astype(jnp.float32)

    mesh = plsc.VectorSubcoreMesh(core_axis_name="c", subcore_axis_name="s")
    run = pl.kernel(
        functools.partial(_gmf_body, D, b_per_w, G),
        mesh=mesh,
        out_type=jax.ShapeDtypeStruct((B,), jnp.float32),
        scratch_types=[
            pltpu.VMEM((b_per_w,), jnp.int32),
            pltpu.VMEM((b_per_w,), jnp.int32),
            pltpu.VMEM((b_per_w, D), jnp.float32),
            pltpu.VMEM((b_per_w, D), jnp.float32),
            pltpu.VMEM((D, _L), jnp.float32),
            pltpu.VMEM((D, _L), jnp.float32),
            pltpu.VMEM((_L,), jnp.float32),
            pltpu.VMEM((b_per_w,), jnp.float32),
            pltpu.SemaphoreType.DMA,
            pltpu.SemaphoreType.DMA,
        ],
        compiler_params=pltpu.CompilerParams(
            needs_layout_passes=False, use_tc_tiling_on_sc=False),
    )
    out = run(u_idx, i_idx, user_table, item_table, wu_b, wi_b, bias_b)
    return out.reshape(B, 1)
